# Initial kernel scaffold; baseline (speedup 1.0000x reference)
import functools
import math

import jax
import jax.numpy as jnp
import numpy as np
from jax import lax
from jax.experimental import pallas as pl
from jax.experimental.pallas import tpu as pltpu
from jax.experimental.pallas import tpu_sc as plsc

SEQ = 2048
DIM = 1024
NUM_HEADS = 16
HEAD_DIM = 64
KNBR = 32
SCALE = 1.0 / math.sqrt(HEAD_DIM)

_J = np.arange(DIM)
PERM = jnp.asarray((_J % NUM_HEADS) * HEAD_DIM + _J // NUM_HEADS, dtype=jnp.int32)


def _mm_kernel(a_ref, w_ref, b_ref, o_ref):
    acc = lax.dot_general(
        a_ref[...], w_ref[...],
        dimension_numbers=(((1,), (1,)), ((), ())),
        preferred_element_type=jnp.float32,
    )
    o_ref[...] = acc + b_ref[0, :][None, :]


def _matmul(a, w, b, bm=512, bn=512):
    m, k = a.shape
    n = w.shape[0]
    b2 = b.reshape(1, n)
    return pl.pallas_call(
        _mm_kernel,
        grid=(m // bm, n // bn),
        in_specs=[
            pl.BlockSpec((bm, k), lambda i, j: (i, 0)),
            pl.BlockSpec((bn, k), lambda i, j: (j, 0)),
            pl.BlockSpec((1, bn), lambda i, j: (0, j)),
        ],
        out_specs=pl.BlockSpec((bm, bn), lambda i, j: (i, j)),
        out_shape=jax.ShapeDtypeStruct((m, n), jnp.float32),
    )(a, w, b2)


_INFO = plsc.get_sparse_core_info()
_NC, _NS = _INFO.num_cores, _INFO.num_subcores
_NW = _NC * _NS
_QPW = SEQ // _NW


def _attn_body(q_hbm, kv_hbm, routes_hbm, out_hbm, idx_v, qrow, kvrows, sbuf,
               orow, sem):
    wid = lax.axis_index("s") * _NC + lax.axis_index("c")
    base = wid * _QPW

    def one_query(il, _):
        i = base + il
        pltpu.sync_copy(routes_hbm.at[i], idx_v)
        cp = pltpu.async_copy(kv_hbm.at[idx_v], kvrows, sem)
        pltpu.sync_copy(q_hbm.at[i], qrow)
        cp.wait()

        def score_j(j, _):
            def dot_d(d, acc):
                qv = qrow[pl.ds(d * 16, 16)]
                kvv = kvrows[j, pl.ds(d * 16, 16)]
                return acc + qv * kvv

            acc = lax.fori_loop(0, HEAD_DIM, dot_d, jnp.zeros(16, jnp.float32),
                                unroll=8)
            sbuf[j, :] = acc * SCALE
            return 0

        lax.fori_loop(0, KNBR, score_j, 0)

        def max_j(j, m):
            return jnp.maximum(m, sbuf[j, :])

        m = lax.fori_loop(0, KNBR, max_j,
                          jnp.full((16,), -jnp.inf, jnp.float32), unroll=4)

        def exp_j(j, s):
            e = jnp.exp(sbuf[j, :] - m)
            sbuf[j, :] = e
            return s + e

        s = lax.fori_loop(0, KNBR, exp_j, jnp.zeros(16, jnp.float32), unroll=4)
        r = 1.0 / s

        def norm_j(j, _):
            sbuf[j, :] = sbuf[j, :] * r
            return 0

        lax.fori_loop(0, KNBR, norm_j, 0, unroll=4)

        def out_d(d, _):
            def acc_j(j, acc):
                return acc + sbuf[j, :] * kvrows[j, pl.ds(DIM + d * 16, 16)]

            acc = lax.fori_loop(0, KNBR, acc_j, jnp.zeros(16, jnp.float32),
                                unroll=8)
            orow[pl.ds(d * 16, 16)] = acc
            return 0

        lax.fori_loop(0, HEAD_DIM, out_d, 0)
        pltpu.sync_copy(orow, out_hbm.at[i])
        return 0

    lax.fori_loop(0, _QPW, one_query, 0)


_sc_attention = functools.partial(
    pl.kernel,
    mesh=plsc.VectorSubcoreMesh(core_axis_name="c", subcore_axis_name="s"),
    out_type=jax.ShapeDtypeStruct((SEQ, DIM), jnp.float32),
    scratch_types=[
        pltpu.VMEM((KNBR,), jnp.int32),
        pltpu.VMEM((DIM,), jnp.float32),
        pltpu.VMEM((KNBR, 2 * DIM), jnp.float32),
        pltpu.VMEM((KNBR, 16), jnp.float32),
        pltpu.VMEM((DIM,), jnp.float32),
        pltpu.SemaphoreType.DMA,
    ],
)(_attn_body)


def kernel(x, W_qkv, b_qkv, W_out, b_out, routes):
    xs = x.reshape(SEQ, DIM)
    wq = jnp.take(W_qkv, PERM, axis=0)
    wk = jnp.take(W_qkv, DIM + PERM, axis=0)
    wv = jnp.take(W_qkv, 2 * DIM + PERM, axis=0)
    bq = jnp.take(b_qkv, PERM)
    bkv = jnp.concatenate([jnp.take(b_qkv, DIM + PERM),
                           jnp.take(b_qkv, 2 * DIM + PERM)])

    q = _matmul(xs, wq, bq)
    kv = _matmul(xs, jnp.concatenate([wk, wv], axis=0), bkv)
    attn = _sc_attention(q, kv, routes)
    out = _matmul(attn, jnp.take(W_out, PERM, axis=1), b_out)
    return out.reshape(1, SEQ, DIM)


# speedup vs baseline: 2.8011x; 2.8011x over previous
import functools
import math

import jax
import jax.numpy as jnp
import numpy as np
from jax import lax
from jax.experimental import pallas as pl
from jax.experimental.pallas import tpu as pltpu
from jax.experimental.pallas import tpu_sc as plsc

SEQ = 2048
DIM = 1024
NUM_HEADS = 16
HEAD_DIM = 64
KNBR = 32
SCALE = 1.0 / math.sqrt(HEAD_DIM)

_J = np.arange(DIM)
PERM = np.asarray((_J % NUM_HEADS) * HEAD_DIM + _J // NUM_HEADS, dtype=np.int32)


def _mm_kernel(a_ref, w_ref, b_ref, o_ref):
    acc = lax.dot_general(
        a_ref[...], w_ref[...],
        dimension_numbers=(((1,), (1,)), ((), ())),
        preferred_element_type=jnp.float32,
    )
    o_ref[...] = acc + b_ref[0, :][None, :]


def _matmul(a, w, b, bm=512, bn=512):
    m, k = a.shape
    n = w.shape[0]
    b2 = b.reshape(1, n)
    return pl.pallas_call(
        _mm_kernel,
        grid=(m // bm, n // bn),
        in_specs=[
            pl.BlockSpec((bm, k), lambda i, j: (i, 0)),
            pl.BlockSpec((bn, k), lambda i, j: (j, 0)),
            pl.BlockSpec((1, bn), lambda i, j: (0, j)),
        ],
        out_specs=pl.BlockSpec((bm, bn), lambda i, j: (i, j)),
        out_shape=jax.ShapeDtypeStruct((m, n), jnp.float32),
    )(a, w, b2)


_NC, _NS = 2, 16
_NW = _NC * _NS
_QPW = SEQ // _NW


def _attn_body(q_hbm, kv_hbm, routes_hbm, out_hbm, idx_v, qrow, kvrows, sbuf,
               orow, sem):
    wid = lax.axis_index("s") * _NC + lax.axis_index("c")
    base = wid * _QPW

    def one_query(il, _):
        i = base + il
        pltpu.sync_copy(routes_hbm.at[i], idx_v)
        cp = pltpu.async_copy(kv_hbm.at[idx_v], kvrows, sem)
        pltpu.sync_copy(q_hbm.at[i], qrow)
        cp.wait()

        def score_j(j, _):
            def dot_d(d, acc):
                qv = qrow[pl.ds(d * 16, 16)]
                kvv = kvrows[j, pl.ds(d * 16, 16)]
                return acc + qv * kvv

            acc = lax.fori_loop(0, HEAD_DIM, dot_d, jnp.zeros(16, jnp.float32),
                                unroll=8)
            sbuf[j, :] = acc * SCALE
            return 0

        lax.fori_loop(0, KNBR, score_j, 0)

        def max_j(j, m):
            return jnp.maximum(m, sbuf[j, :])

        m = lax.fori_loop(0, KNBR, max_j,
                          jnp.full((16,), -jnp.inf, jnp.float32), unroll=4)

        def exp_j(j, s):
            e = jnp.exp(sbuf[j, :] - m)
            sbuf[j, :] = e
            return s + e

        s = lax.fori_loop(0, KNBR, exp_j, jnp.zeros(16, jnp.float32), unroll=4)
        r = 1.0 / s

        def norm_j(j, _):
            sbuf[j, :] = sbuf[j, :] * r
            return 0

        lax.fori_loop(0, KNBR, norm_j, 0, unroll=4)

        def out_d(d, _):
            def acc_j(j, acc):
                return acc + sbuf[j, :] * kvrows[j, pl.ds(DIM + d * 16, 16)]

            acc = lax.fori_loop(0, KNBR, acc_j, jnp.zeros(16, jnp.float32),
                                unroll=8)
            orow[pl.ds(d * 16, 16)] = acc
            return 0

        lax.fori_loop(0, HEAD_DIM, out_d, 0)
        pltpu.sync_copy(orow, out_hbm.at[i])
        return 0

    lax.fori_loop(0, _QPW, one_query, 0)


def _sc_attention(q, kv, routes):
    attn_fn = pl.kernel(
        _attn_body,
        mesh=plsc.VectorSubcoreMesh(core_axis_name="c", subcore_axis_name="s"),
        out_type=jax.ShapeDtypeStruct((SEQ, DIM), jnp.float32),
        scratch_types=[
            pltpu.VMEM((KNBR,), jnp.int32),
            pltpu.VMEM((DIM,), jnp.float32),
            pltpu.VMEM((KNBR, 2 * DIM), jnp.float32),
            pltpu.VMEM((KNBR, 16), jnp.float32),
            pltpu.VMEM((DIM,), jnp.float32),
            pltpu.SemaphoreType.DMA,
        ],
    )
    return attn_fn(q, kv, routes)


def kernel(x, W_qkv, b_qkv, W_out, b_out, routes):
    xs = x.reshape(SEQ, DIM)
    wq = jnp.take(W_qkv, PERM, axis=0)
    wk = jnp.take(W_qkv, DIM + PERM, axis=0)
    wv = jnp.take(W_qkv, 2 * DIM + PERM, axis=0)
    bq = jnp.take(b_qkv, PERM)
    bkv = jnp.concatenate([jnp.take(b_qkv, DIM + PERM),
                           jnp.take(b_qkv, 2 * DIM + PERM)])

    q = _matmul(xs, wq, bq)
    kv = _matmul(xs, jnp.concatenate([wk, wv], axis=0), bkv)
    attn = _sc_attention(q, kv, routes)
    out = _matmul(attn, jnp.take(W_out, PERM, axis=1), b_out)
    return out.reshape(1, SEQ, DIM)


# speedup vs baseline: 4.5393x; 1.6205x over previous
import functools
import math

import jax
import jax.numpy as jnp
import numpy as np
from jax import lax
from jax.experimental import pallas as pl
from jax.experimental.pallas import tpu as pltpu
from jax.experimental.pallas import tpu_sc as plsc

SEQ = 2048
DIM = 1024
NUM_HEADS = 16
HEAD_DIM = 64
KNBR = 32
SCALE = 1.0 / math.sqrt(HEAD_DIM)

_J = np.arange(DIM)
PERM = np.asarray((_J % NUM_HEADS) * HEAD_DIM + _J // NUM_HEADS, dtype=np.int32)


def _mm_kernel(a_ref, w_ref, b_ref, o_ref):
    acc = lax.dot_general(
        a_ref[...], w_ref[...],
        dimension_numbers=(((1,), (1,)), ((), ())),
        preferred_element_type=jnp.float32,
    )
    o_ref[...] = acc + b_ref[0, :][None, :]


def _matmul(a, w, b, bm=512, bn=512):
    m, k = a.shape
    n = w.shape[0]
    b2 = b.reshape(1, n)
    return pl.pallas_call(
        _mm_kernel,
        grid=(m // bm, n // bn),
        in_specs=[
            pl.BlockSpec((bm, k), lambda i, j: (i, 0)),
            pl.BlockSpec((bn, k), lambda i, j: (j, 0)),
            pl.BlockSpec((1, bn), lambda i, j: (0, j)),
        ],
        out_specs=pl.BlockSpec((bm, bn), lambda i, j: (i, j)),
        out_shape=jax.ShapeDtypeStruct((m, n), jnp.float32),
    )(a, w, b2)


_NC, _NS = 2, 16
_NW = _NC * _NS
_QPW = SEQ // _NW
_JT = 8
_DT = 8


def _attn_body(q_hbm, k_hbm, v_hbm, routes_hbm, out_hbm,
               idx0, idx1, kbuf0, kbuf1, qbuf0, qbuf1, vbuf, sbuf, orow,
               semk, semv):
    wid = lax.axis_index("s") * _NC + lax.axis_index("c")
    base = wid * _QPW

    idxs = (idx0, idx1)
    kbufs = (kbuf0, kbuf1)
    qbufs = (qbuf0, qbuf1)

    pltpu.sync_copy(routes_hbm.at[base], idx0)
    pltpu.async_copy(k_hbm.at[idx0], kbuf0, semk)
    pltpu.async_copy(q_hbm.at[base], qbuf0, semk)

    def one_query(slot, i, inext):
        idxc, kb, qb = idxs[slot], kbufs[slot], qbufs[slot]
        idxn, kbn, qbn = idxs[1 - slot], kbufs[1 - slot], qbufs[1 - slot]

        pltpu.make_async_copy(k_hbm.at[idxc], kb, semk).wait()
        pltpu.make_async_copy(q_hbm.at[i], qb, semk).wait()

        cpv = pltpu.async_copy(v_hbm.at[idxc], vbuf, semv)

        pltpu.sync_copy(routes_hbm.at[inext], idxn)
        pltpu.async_copy(k_hbm.at[idxn], kbn, semk)
        pltpu.async_copy(q_hbm.at[inext], qbn, semk)

        for jt in range(KNBR // _JT):
            def dot_d(d, accs, jt=jt):
                qv = qb[pl.ds(d * 16, 16)]
                return tuple(
                    accs[u] + qv * kb[jt * _JT + u, pl.ds(d * 16, 16)]
                    for u in range(_JT))

            accs = lax.fori_loop(
                0, HEAD_DIM, dot_d,
                tuple(jnp.zeros(16, jnp.float32) for _ in range(_JT)),
                unroll=4)
            for u in range(_JT):
                sbuf[jt * _JT + u, :] = accs[u]

        def max_j(j, m):
            return jnp.maximum(m, sbuf[j, :])

        m = lax.fori_loop(0, KNBR, max_j,
                          jnp.full((16,), -jnp.inf, jnp.float32), unroll=4)

        def exp_j(j, s):
            e = jnp.exp(sbuf[j, :] - m)
            sbuf[j, :] = e
            return s + e

        s = lax.fori_loop(0, KNBR, exp_j, jnp.zeros(16, jnp.float32), unroll=4)
        r = 1.0 / s

        cpv.wait()

        for dt in range(HEAD_DIM // _DT):
            def acc_j(j, accs, dt=dt):
                wv = sbuf[j, :]
                return tuple(
                    accs[u] + wv * vbuf[j, pl.ds((dt * _DT + u) * 16, 16)]
                    for u in range(_DT))

            accs = lax.fori_loop(
                0, KNBR, acc_j,
                tuple(jnp.zeros(16, jnp.float32) for _ in range(_DT)),
                unroll=4)
            for u in range(_DT):
                orow[pl.ds((dt * _DT + u) * 16, 16)] = accs[u] * r

        pltpu.sync_copy(orow, out_hbm.at[i])

    def pair(h, _):
        i0 = base + 2 * h
        one_query(0, i0, i0 + 1)
        inext = jnp.where(2 * h + 2 < _QPW, i0 + 2, base)
        one_query(1, i0 + 1, inext)
        return 0

    lax.fori_loop(0, _QPW // 2, pair, 0)

    pltpu.make_async_copy(k_hbm.at[idx0], kbuf0, semk).wait()
    pltpu.make_async_copy(q_hbm.at[base], qbuf0, semk).wait()


def _sc_attention(q, k, v, routes):
    attn_fn = pl.kernel(
        _attn_body,
        mesh=plsc.VectorSubcoreMesh(core_axis_name="c", subcore_axis_name="s"),
        out_type=jax.ShapeDtypeStruct((SEQ, DIM), jnp.float32),
        scratch_types=[
            pltpu.VMEM((KNBR,), jnp.int32),
            pltpu.VMEM((KNBR,), jnp.int32),
            pltpu.VMEM((KNBR, DIM), jnp.float32),
            pltpu.VMEM((KNBR, DIM), jnp.float32),
            pltpu.VMEM((DIM,), jnp.float32),
            pltpu.VMEM((DIM,), jnp.float32),
            pltpu.VMEM((KNBR, DIM), jnp.float32),
            pltpu.VMEM((KNBR, 16), jnp.float32),
            pltpu.VMEM((DIM,), jnp.float32),
            pltpu.SemaphoreType.DMA,
            pltpu.SemaphoreType.DMA,
        ],
    )
    return attn_fn(q, k, v, routes)


def kernel(x, W_qkv, b_qkv, W_out, b_out, routes):
    xs = x.reshape(SEQ, DIM)
    wq = jnp.take(W_qkv, PERM, axis=0) * SCALE
    wk = jnp.take(W_qkv, DIM + PERM, axis=0)
    wv = jnp.take(W_qkv, 2 * DIM + PERM, axis=0)
    bq = jnp.take(b_qkv, PERM) * SCALE
    bk = jnp.take(b_qkv, DIM + PERM)
    bv = jnp.take(b_qkv, 2 * DIM + PERM)

    q = _matmul(xs, wq, bq)
    k = _matmul(xs, wk, bk)
    v = _matmul(xs, wv, bv)
    attn = _sc_attention(q, k, v, routes)
    out = _matmul(attn, jnp.take(W_out, PERM, axis=1), b_out)
    return out.reshape(1, SEQ, DIM)


# speedup vs baseline: 4.5448x; 1.0012x over previous
import functools
import math

import jax
import jax.numpy as jnp
import numpy as np
from jax import lax
from jax.experimental import pallas as pl
from jax.experimental.pallas import tpu as pltpu
from jax.experimental.pallas import tpu_sc as plsc

SEQ = 2048
DIM = 1024
NUM_HEADS = 16
HEAD_DIM = 64
KNBR = 32
SCALE = 1.0 / math.sqrt(HEAD_DIM)

_J = np.arange(DIM)
PERM = np.asarray((_J % NUM_HEADS) * HEAD_DIM + _J // NUM_HEADS, dtype=np.int32)


def _mm_kernel(a_ref, w_ref, b_ref, o_ref):
    acc = lax.dot_general(
        a_ref[...], w_ref[...],
        dimension_numbers=(((1,), (1,)), ((), ())),
        preferred_element_type=jnp.float32,
    )
    o_ref[...] = (acc + b_ref[0, :][None, :]).astype(o_ref.dtype)


def _matmul(a, w, b, bm=512, bn=512, out_dtype=jnp.float32):
    m, k = a.shape
    n = w.shape[0]
    b2 = b.reshape(1, n)
    return pl.pallas_call(
        _mm_kernel,
        grid=(m // bm, n // bn),
        in_specs=[
            pl.BlockSpec((bm, k), lambda i, j: (i, 0)),
            pl.BlockSpec((bn, k), lambda i, j: (j, 0)),
            pl.BlockSpec((1, bn), lambda i, j: (0, j)),
        ],
        out_specs=pl.BlockSpec((bm, bn), lambda i, j: (i, j)),
        out_shape=jax.ShapeDtypeStruct((m, n), out_dtype),
    )(a, w, b2)


_NC, _NS = 2, 16
_NW = _NC * _NS
_QPW = SEQ // _NW
_JT = 8
_OT = 8


def _attn_body(q_hbm, k_hbm, v_hbm, routes_hbm, out_hbm,
               idx0, idx1, kbuf0, kbuf1, qbuf0, qbuf1, vbuf, sbuf, orow,
               semk, semv):
    wid = lax.axis_index("s") * _NC + lax.axis_index("c")
    base = wid * _QPW

    idxs = (idx0, idx1)
    kbufs = (kbuf0, kbuf1)
    qbufs = (qbuf0, qbuf1)

    pltpu.sync_copy(routes_hbm.at[base], idx0)
    pltpu.async_copy(k_hbm.at[idx0], kbuf0, semk)
    pltpu.async_copy(q_hbm.at[base], qbuf0, semk)

    def one_query(slot, i, inext):
        idxc, kb, qb = idxs[slot], kbufs[slot], qbufs[slot]
        idxn, kbn, qbn = idxs[1 - slot], kbufs[1 - slot], qbufs[1 - slot]

        pltpu.make_async_copy(k_hbm.at[idxc], kb, semk).wait()
        pltpu.make_async_copy(q_hbm.at[i], qb, semk).wait()

        cpv = pltpu.async_copy(v_hbm.at[idxc], vbuf, semv)

        pltpu.sync_copy(routes_hbm.at[inext], idxn)
        pltpu.async_copy(k_hbm.at[idxn], kbn, semk)
        pltpu.async_copy(q_hbm.at[inext], qbn, semk)

        for jt in range(KNBR // _JT):
            def dot_d(d, accs, jt=jt):
                qv = qb[pl.ds(d * 16, 16)]
                out = []
                for u in range(_JT):
                    out.append(accs[u] + qv * kb[jt * _JT + u, pl.ds(d * 16, 16)])
                return tuple(out)

            accs = lax.fori_loop(
                0, HEAD_DIM, dot_d,
                tuple(jnp.zeros(16, jnp.float32) for _ in range(_JT)),
                unroll=4)
            for u in range(_JT):
                sbuf[jt * _JT + u, :] = accs[u]

        def max_j(j, m):
            return jnp.maximum(m, sbuf[j, :])

        m = lax.fori_loop(0, KNBR, max_j,
                          jnp.full((16,), -jnp.inf, jnp.float32), unroll=4)

        def exp_j(j, s):
            e = jnp.exp(sbuf[j, :] - m)
            sbuf[j, :] = e
            return s + e

        s = lax.fori_loop(0, KNBR, exp_j, jnp.zeros(16, jnp.float32), unroll=4)
        r = 1.0 / s

        cpv.wait()

        for pt in range(HEAD_DIM // _OT):
            def acc_j(j, accs, pt=pt):
                wv = sbuf[j, :]
                out = []
                for t in range(_OT):
                    out.append(
                        accs[t] + wv * vbuf[j, pl.ds((pt * _OT + t) * 16, 16)])
                return tuple(out)

            accs = lax.fori_loop(
                0, KNBR, acc_j,
                tuple(jnp.zeros(16, jnp.float32) for _ in range(_OT)),
                unroll=4)
            for t in range(_OT):
                orow[pl.ds((pt * _OT + t) * 16, 16)] = accs[t] * r

        pltpu.sync_copy(orow, out_hbm.at[i])

    def pair(h, _):
        i0 = base + 2 * h
        one_query(0, i0, i0 + 1)
        inext = jnp.where(2 * h + 2 < _QPW, i0 + 2, base)
        one_query(1, i0 + 1, inext)
        return 0

    lax.fori_loop(0, _QPW // 2, pair, 0)

    pltpu.make_async_copy(k_hbm.at[idx0], kbuf0, semk).wait()
    pltpu.make_async_copy(q_hbm.at[base], qbuf0, semk).wait()


def _sc_attention(q, k, v, routes):
    attn_fn = pl.kernel(
        _attn_body,
        mesh=plsc.VectorSubcoreMesh(core_axis_name="c", subcore_axis_name="s"),
        out_type=jax.ShapeDtypeStruct((SEQ, DIM), jnp.float32),
        scratch_types=[
            pltpu.VMEM((KNBR,), jnp.int32),
            pltpu.VMEM((KNBR,), jnp.int32),
            pltpu.VMEM((KNBR, DIM), jnp.float32),
            pltpu.VMEM((KNBR, DIM), jnp.float32),
            pltpu.VMEM((DIM,), jnp.float32),
            pltpu.VMEM((DIM,), jnp.float32),
            pltpu.VMEM((KNBR, DIM), jnp.float32),
            pltpu.VMEM((KNBR, 16), jnp.float32),
            pltpu.VMEM((DIM,), jnp.float32),
            pltpu.SemaphoreType.DMA,
            pltpu.SemaphoreType.DMA,
        ],
    )
    return attn_fn(q, k, v, routes)


def kernel(x, W_qkv, b_qkv, W_out, b_out, routes):
    xs = x.reshape(SEQ, DIM)
    wq = jnp.take(W_qkv, PERM, axis=0) * SCALE
    wk = jnp.take(W_qkv, DIM + PERM, axis=0)
    wv = jnp.take(W_qkv, 2 * DIM + PERM, axis=0)
    bq = jnp.take(b_qkv, PERM) * SCALE
    bk = jnp.take(b_qkv, DIM + PERM)
    bv = jnp.take(b_qkv, 2 * DIM + PERM)

    q = _matmul(xs, wq, bq)
    k = _matmul(xs, wk, bk)
    v = _matmul(xs, wv, bv)
    attn = _sc_attention(q, k, v, routes)
    out = _matmul(attn, jnp.take(W_out, PERM, axis=1), b_out)
    return out.reshape(1, SEQ, DIM)
